# baseline (speedup 1.0000x reference)
import jax
import jax.numpy as jnp
from jax import lax
from jax.experimental import pallas as pl
from jax.experimental.pallas import tpu as pltpu
from jax.experimental.pallas import tpu_sc as plsc

_N = 10000
_K = 5000
_D = 512

_NC = 2
_NS = 16
_NW = _NC * _NS
_ZCH = 320
_HALF = 160
_W = 176
_SUB = 88
_ZR = 40
_KPAD = 5024


def _unpool_body(x_hbm, idx_hbm, z_hbm, out_hbm, idx_all, xv, zrows, idx2, sem):
    wid = lax.axis_index("s") * _NC + lax.axis_index("c")

    pltpu.sync_copy(idx_hbm, idx_all.at[pl.ds(0, _K)])
    sent = jnp.full((16,), _N, jnp.int32)
    idx_all[pl.ds(_K, 16)] = sent
    idx_all[pl.ds(_KPAD - 16, 16)] = sent
    pltpu.sync_copy(z_hbm, zrows)

    def _lower_bound(val):
        r = jnp.zeros((16,), jnp.int32) + val
        pos = jnp.zeros((16,), jnp.int32)
        for step in (4096, 2048, 1024, 512, 256, 128, 64, 32, 16, 8, 4, 2, 1):
            probe = jnp.minimum(pos + step - 1, _KPAD - 1)
            v = plsc.load_gather(idx_all, [probe])
            pos = jnp.where(v < r, pos + step, pos)
        return pos[0]

    nb = pl.multiple_of(jnp.minimum(wid * _ZCH, _N - _ZCH), 8)
    for u in range(_ZCH // _ZR):
        pltpu.sync_copy(zrows, out_hbm.at[pl.ds(nb + u * _ZR, _ZR)])

    for h in range(2):
        hb = nb + h * _HALF
        jlo = _lower_bound(hb)
        jhi = _lower_bound(hb + _HALF)

        @pl.when(jhi > jlo)
        def _scatter_half():
            jcl = pl.multiple_of(jnp.minimum((jlo // 8) * 8, _K - _W), 8)
            pltpu.sync_copy(idx_hbm.at[pl.ds(jcl, _SUB)], idx2.at[0])
            pltpu.sync_copy(idx_hbm.at[pl.ds(jcl + _SUB, _SUB)], idx2.at[1])
            pltpu.sync_copy(x_hbm.at[pl.ds(jcl, _W)], xv)
            cp0 = pltpu.async_copy(xv.at[pl.ds(0, _SUB)], out_hbm.at[idx2.at[0]], sem)
            cp1 = pltpu.async_copy(xv.at[pl.ds(_SUB, _SUB)], out_hbm.at[idx2.at[1]], sem)
            cp0.wait()
            cp1.wait()


_mesh = plsc.VectorSubcoreMesh(
    core_axis_name="c", subcore_axis_name="s", num_cores=_NC, num_subcores=_NS
)
_unpool = pl.kernel(
    _unpool_body,
    out_type=jax.ShapeDtypeStruct((_N, _D), jnp.float32),
    mesh=_mesh,
    compiler_params=pltpu.CompilerParams(needs_layout_passes=False),
    scratch_types=[
        pltpu.VMEM((_KPAD,), jnp.int32),
        pltpu.VMEM((_W, _D), jnp.float32),
        pltpu.VMEM((_ZR, _D), jnp.float32),
        pltpu.VMEM((2, _SUB), jnp.int32),
        pltpu.SemaphoreType.DMA,
    ],
)

_CPROWS = 400


def _copy_body(a_ref, out_ref):
    out_ref[...] = a_ref[...]


_copy = pl.pallas_call(
    _copy_body,
    grid=(_N // _CPROWS,),
    in_specs=[pl.BlockSpec((_CPROWS, _N), lambda i: (i, 0))],
    out_specs=pl.BlockSpec((_CPROWS, _N), lambda i: (i, 0)),
    out_shape=jax.ShapeDtypeStruct((_N, _N), jnp.float32),
    compiler_params=pltpu.CompilerParams(vmem_limit_bytes=100 * 1024 * 1024),
)


def kernel(A, X, idx):
    zsrc = jnp.zeros((_ZR, _D), jnp.float32)
    new_X = _unpool(X, idx.astype(jnp.int32), zsrc)
    return (_copy(A), new_X)


# speedup vs baseline: 1.0150x; 1.0150x over previous
import jax
import jax.numpy as jnp
from jax import lax
from jax.experimental import pallas as pl
from jax.experimental.pallas import tpu as pltpu
from jax.experimental.pallas import tpu_sc as plsc
from jax._src.pallas import mpmd as _mpmd

_N = 10000
_K = 5000
_D = 512

_NC = 2
_NS = 16
_NW = _NC * _NS
_CH = 160
_SUB = 80


def _scatter_body(zeros_hbm, x_hbm, idx_hbm, out_hbm, idxv, xv, sem):
    del zeros_hbm
    wid = lax.axis_index("s") * _NC + lax.axis_index("c")
    base = jnp.minimum(wid * _CH, _K - _CH)
    pltpu.sync_copy(idx_hbm.at[pl.ds(base, _SUB)], idxv.at[0])
    pltpu.sync_copy(idx_hbm.at[pl.ds(base + _SUB, _SUB)], idxv.at[1])
    pltpu.sync_copy(x_hbm.at[pl.ds(base, _CH)], xv)
    cp0 = pltpu.async_copy(xv.at[pl.ds(0, _SUB)], out_hbm.at[idxv.at[0]], sem)
    cp1 = pltpu.async_copy(xv.at[pl.ds(_SUB, _SUB)], out_hbm.at[idxv.at[1]], sem)
    cp0.wait()
    cp1.wait()


_mesh = plsc.VectorSubcoreMesh(
    core_axis_name="c", subcore_axis_name="s", num_cores=_NC, num_subcores=_NS
)
_scatter = _mpmd._mpmd_map(
    [(_mesh, _scatter_body)],
    jax.ShapeDtypeStruct((_N, _D), jnp.float32),
    input_output_aliases={0: 0},
    scratch_types=[
        pltpu.VMEM((2, _SUB), jnp.int32),
        pltpu.VMEM((_CH, _D), jnp.float32),
        pltpu.SemaphoreType.DMA,
    ],
    cost_estimate=pl.CostEstimate(
        flops=0, bytes_accessed=400 * 1024 * 1024, transcendentals=0
    ),
)


_CPROWS = 400


def _copy_body(a_ref, out_ref):
    out_ref[...] = a_ref[...]


_copy = pl.pallas_call(
    _copy_body,
    grid=(_N // _CPROWS,),
    in_specs=[pl.BlockSpec((_CPROWS, _N), lambda i: (i, 0))],
    out_specs=pl.BlockSpec((_CPROWS, _N), lambda i: (i, 0)),
    out_shape=jax.ShapeDtypeStruct((_N, _N), jnp.float32),
    compiler_params=pltpu.CompilerParams(vmem_limit_bytes=192 * 1024 * 1024),
)


def kernel(A, X, idx):
    zeros = jnp.zeros((A.shape[0], X.shape[1]), dtype=X.dtype)
    new_X = _scatter(zeros, X, idx.astype(jnp.int32))
    return (_copy(A), new_X)
